# Initial kernel scaffold; baseline (speedup 1.0000x reference)
import jax
import jax.numpy as jnp
from jax.experimental import pallas as pl


def kernel(x, edge_index, batch, W, b):
    raise NotImplementedError("write your pallas kernel here")


# speedup vs baseline: 59.6612x; 59.6612x over previous
import functools

import jax
import jax.numpy as jnp
from jax import lax
from jax.experimental import pallas as pl
from jax.experimental.pallas import tpu as pltpu
from jax.experimental.pallas import tpu_sc as plsc

_N = 10000
_E = 320000
_D = 128
_G = 64

_NT = 16
_PT = 640
_NPN = _NT * _PT
_EPT = _E // _NT
_ROWS = 157
_EPAD = _ROWS * 128
_VALID_VECS = _EPT // 16


def _matvec_body(x_ref, w_ref, o_ref):
    i = pl.program_id(0)
    h = jnp.dot(x_ref[...], w_ref[...], preferred_element_type=jnp.float32)
    gid = i * _PT + lax.broadcasted_iota(jnp.int32, (_PT, 1), 0)
    o_ref[...] = jnp.where(gid < _N, h, 0.0)


def _tc_matvec(x, W):
    return pl.pallas_call(
        _matvec_body,
        grid=(_NPN // _PT,),
        in_specs=[
            pl.BlockSpec((_PT, _D), lambda i: (i, 0)),
            pl.BlockSpec((_D, 1), lambda i: (0, 0)),
        ],
        out_specs=pl.BlockSpec((_PT, 1), lambda i: (i, 0)),
        out_shape=jax.ShapeDtypeStruct((_NPN, 1), jnp.float32),
    )(x, W)


def _softmax_body(s_ref, b_ref, o_ref):
    s = s_ref[...]
    b = b_ref[...]
    gids = lax.broadcasted_iota(jnp.int32, (_NPN, _G), 1)
    hit = b == gids
    oh = hit.astype(jnp.float32)
    masked = jnp.where(hit, s, -1e30)
    smax = jnp.max(masked, axis=0)
    smax_i = jnp.dot(oh, smax[:, None], preferred_element_type=jnp.float32)
    ex = jnp.exp(s - smax_i)
    ssum = lax.dot_general(oh, ex, (((0,), (0,)), ((), ())),
                           preferred_element_type=jnp.float32)
    ssum_i = jnp.dot(oh, ssum, preferred_element_type=jnp.float32)
    o_ref[...] = ex / (ssum_i + 1e-16)


def _tc_softmax(sc_raw, batchp):
    return pl.pallas_call(
        _softmax_body,
        out_shape=jax.ShapeDtypeStruct((_NPN, 1), jnp.float32),
    )(sc_raw, batchp)


def _rsqrt16(d):
    i = plsc.bitcast(d, jnp.int32)
    i = jnp.int32(0x5F3759DF) - lax.shift_right_logical(i, 1)
    y = plsc.bitcast(i, jnp.float32)
    for _ in range(3):
        y = y * (1.5 - 0.5 * d * y * y)
    return y


def _sc_body(hp, srcp, dstp, scores,
             dstv, srcv, valv, gfull, hown, dego, diso, selfo, gowno,
             sowno, sco, zer, deg_s, g_s, s_s):
    s = lax.axis_index("s")
    base = s * _PT

    def _zero(i, c):
        zer[pl.ds(i * 16, 16)] = jnp.zeros((16,), jnp.float32)
        return c
    lax.fori_loop(0, _PT // 16, _zero, 0)

    pltpu.sync_copy(zer, deg_s.at[pl.ds(base, _PT)])
    pltpu.sync_copy(zer, s_s.at[pl.ds(base, _PT)])
    pltpu.sync_copy(dstp.at[s], dstv)
    pltpu.sync_copy(srcp.at[pl.ds(s * _EPAD, _EPAD)], srcv)
    pltpu.sync_copy(hp.at[pl.ds(base, _PT)], hown)

    def _fill_ones(v, c):
        val = jnp.where(v < _VALID_VECS, 1.0, 0.0).astype(jnp.float32)
        valv[pl.ds(v * 16, 16)] = jnp.zeros((16,), jnp.float32) + val
        return c
    lax.fori_loop(0, _EPAD // 16, _fill_ones, 0)

    plsc.subcore_barrier()

    def _deg_scat(r, c):
        pltpu.sync_copy(valv.at[pl.ds(r * 128, 128)],
                        deg_s.at[dstv.at[r]], add=True)
        return c
    lax.fori_loop(0, _ROWS, _deg_scat, 0)
    plsc.subcore_barrier()

    pltpu.sync_copy(deg_s.at[pl.ds(base, _PT)], dego)

    def _norm(i, c):
        sl = pl.ds(i * 16, 16)
        d = dego[sl] + 1.0
        y = _rsqrt16(d)
        diso[sl] = y
        selfo[sl] = hown[sl] / d
        gowno[sl] = y * hown[sl]
        return c
    lax.fori_loop(0, _PT // 16, _norm, 0)

    pltpu.sync_copy(gowno, g_s.at[pl.ds(base, _PT)])
    plsc.subcore_barrier()
    pltpu.sync_copy(g_s, gfull)

    def _gather(v, c):
        sl = pl.ds(v * 16, 16)
        valv[sl] = plsc.load_gather(gfull, [srcv[sl]])
        return c
    lax.fori_loop(0, _EPAD // 16, _gather, 0)

    def _msg_scat(r, c):
        pltpu.sync_copy(valv.at[pl.ds(r * 128, 128)],
                        s_s.at[dstv.at[r]], add=True)
        return c
    lax.fori_loop(0, _ROWS, _msg_scat, 0)
    plsc.subcore_barrier()

    pltpu.sync_copy(s_s.at[pl.ds(base, _PT)], sowno)

    def _score(i, c):
        sl = pl.ds(i * 16, 16)
        sc = diso[sl] * sowno[sl] + selfo[sl]
        gid = base + i * 16 + lax.iota(jnp.int32, 16)
        sco[sl] = jnp.where(gid < _N, sc, -1e30)
        return c
    lax.fori_loop(0, _PT // 16, _score, 0)
    pltpu.sync_copy(sco, scores.at[pl.ds(base, _PT)])


_sc_call = functools.partial(
    pl.kernel,
    out_type=jax.ShapeDtypeStruct((_NPN,), jnp.float32),
    mesh=plsc.VectorSubcoreMesh(core_axis_name="c", subcore_axis_name="s"),
    compiler_params=pltpu.CompilerParams(needs_layout_passes=False),
    scratch_types=[
        pltpu.VMEM((_ROWS, 128), jnp.int32),
        pltpu.VMEM((_EPAD,), jnp.int32),
        pltpu.VMEM((_EPAD,), jnp.float32),
        pltpu.VMEM((_NPN,), jnp.float32),
        pltpu.VMEM((_PT,), jnp.float32),
        pltpu.VMEM((_PT,), jnp.float32),
        pltpu.VMEM((_PT,), jnp.float32),
        pltpu.VMEM((_PT,), jnp.float32),
        pltpu.VMEM((_PT,), jnp.float32),
        pltpu.VMEM((_PT,), jnp.float32),
        pltpu.VMEM((_PT,), jnp.float32),
        pltpu.VMEM((_PT,), jnp.float32),
        pltpu.VMEM_SHARED((_NPN,), jnp.float32),
        pltpu.VMEM_SHARED((_NPN,), jnp.float32),
        pltpu.VMEM_SHARED((_NPN,), jnp.float32),
    ],
)(_sc_body)


def _pad_edges(idx, flat):
    per = _E // _NT
    npad = _EPAD - per
    pad = 10000 + (jnp.arange(_NT * npad, dtype=jnp.int32) % 240)
    pad = pad.reshape(_NT, npad)
    out = jnp.concatenate([idx.reshape(_NT, per), pad], axis=1)
    return out.reshape(_NT * _EPAD) if flat else out.reshape(_NT, _ROWS, 128)


def kernel(x, edge_index, batch, W, b):
    h = _tc_matvec(x, W).reshape(_NPN)
    srcp = _pad_edges(edge_index[0], flat=True)
    dstp = _pad_edges(edge_index[1], flat=False)
    sc_raw = _sc_call(h, srcp, dstp)
    batchp = jnp.concatenate(
        [batch, jnp.full((_NPN - _N,), -1, jnp.int32)]).reshape(_NPN, 1)
    scores = _tc_softmax(sc_raw.reshape(_NPN, 1), batchp)
    scores = scores[:_N].reshape(_N, 1)
    perm = jnp.arange(_N, dtype=jnp.int32)
    return (x, edge_index, batch, perm, scores)


# speedup vs baseline: 60.6978x; 1.0174x over previous
import functools

import jax
import jax.numpy as jnp
from jax import lax
from jax.experimental import pallas as pl
from jax.experimental.pallas import tpu as pltpu
from jax.experimental.pallas import tpu_sc as plsc

_N = 10000
_E = 320000
_D = 128
_G = 64

_NT = 16
_PT = 640
_NPN = _NT * _PT
_EPT = _E // _NT
_ROWS = 157
_EPAD = _ROWS * 128
_VALID_VECS = _EPT // 16


def _matvec_body(x_ref, w_ref, o_ref):
    i = pl.program_id(0)
    h = jnp.dot(x_ref[...], w_ref[...], preferred_element_type=jnp.float32)
    gid = i * _PT + lax.broadcasted_iota(jnp.int32, (_PT, 1), 0)
    o_ref[...] = jnp.where(gid < _N, h, 0.0)


def _tc_matvec(x, W):
    return pl.pallas_call(
        _matvec_body,
        grid=(_NPN // _PT,),
        in_specs=[
            pl.BlockSpec((_PT, _D), lambda i: (i, 0)),
            pl.BlockSpec((_D, 1), lambda i: (0, 0)),
        ],
        out_specs=pl.BlockSpec((_PT, 1), lambda i: (i, 0)),
        out_shape=jax.ShapeDtypeStruct((_NPN, 1), jnp.float32),
    )(x, W)


def _softmax_body(s_ref, b_ref, o_ref):
    s = s_ref[...]
    b = b_ref[...]
    gids = lax.broadcasted_iota(jnp.int32, (_NPN, _G), 1)
    hit = b == gids
    oh = hit.astype(jnp.float32)
    masked = jnp.where(hit, s, -1e30)
    smax = jnp.max(masked, axis=0)
    smax_i = jnp.dot(oh, smax[:, None], preferred_element_type=jnp.float32)
    ex = jnp.exp(s - smax_i)
    ssum = lax.dot_general(oh, ex, (((0,), (0,)), ((), ())),
                           preferred_element_type=jnp.float32)
    ssum_i = jnp.dot(oh, ssum, preferred_element_type=jnp.float32)
    o_ref[...] = ex / (ssum_i + 1e-16)


def _tc_softmax(sc_raw, batchp):
    return pl.pallas_call(
        _softmax_body,
        out_shape=jax.ShapeDtypeStruct((_NPN, 1), jnp.float32),
    )(sc_raw, batchp)


def _rsqrt16(d):
    i = plsc.bitcast(d, jnp.int32)
    i = jnp.int32(0x5F3759DF) - lax.shift_right_logical(i, 1)
    y = plsc.bitcast(i, jnp.float32)
    for _ in range(3):
        y = y * (1.5 - 0.5 * d * y * y)
    return y


def _sc_body(hp, srcp, dstp, scores,
             dstv, srcv, valv, gfull, hown, dego, diso, selfo, gowno,
             sowno, sco, zer, deg_s, g_s, s_s):
    s = lax.axis_index("s")
    base = s * _PT

    def _zero(i, c):
        zer[pl.ds(i * 16, 16)] = jnp.zeros((16,), jnp.float32)
        return c
    lax.fori_loop(0, _PT // 16, _zero, 0)

    pltpu.sync_copy(zer, deg_s.at[pl.ds(base, _PT)])
    pltpu.sync_copy(zer, s_s.at[pl.ds(base, _PT)])
    pltpu.sync_copy(dstp.at[s], dstv)
    pltpu.sync_copy(srcp.at[pl.ds(s * _EPAD, _EPAD)], srcv)
    pltpu.sync_copy(hp.at[pl.ds(base, _PT)], hown)

    def _fill_ones(v, c):
        val = jnp.where(v < _VALID_VECS, 1.0, 0.0).astype(jnp.float32)
        valv[pl.ds(v * 16, 16)] = jnp.zeros((16,), jnp.float32) + val
        return c
    lax.fori_loop(0, _EPAD // 16, _fill_ones, 0)

    plsc.subcore_barrier()

    def _deg_scat(r, c):
        pltpu.sync_copy(valv.at[pl.ds(r * 128, 128)],
                        deg_s.at[dstv.at[r]], add=True)
        return c
    lax.fori_loop(0, _ROWS, _deg_scat, 0)
    plsc.subcore_barrier()

    pltpu.sync_copy(deg_s.at[pl.ds(base, _PT)], dego)

    def _norm(i, c):
        sl = pl.ds(i * 16, 16)
        d = dego[sl] + 1.0
        y = _rsqrt16(d)
        diso[sl] = y
        selfo[sl] = hown[sl] / d
        gowno[sl] = y * hown[sl]
        return c
    lax.fori_loop(0, _PT // 16, _norm, 0)

    pltpu.sync_copy(gowno, g_s.at[pl.ds(base, _PT)])
    plsc.subcore_barrier()
    pltpu.sync_copy(g_s, gfull)

    def _gather(v, c):
        sl = pl.ds(v * 16, 16)
        valv[sl] = plsc.load_gather(gfull, [srcv[sl]])
        return c
    lax.fori_loop(0, _EPAD // 16, _gather, 0)

    def _msg_scat(r, c):
        pltpu.sync_copy(valv.at[pl.ds(r * 128, 128)],
                        s_s.at[dstv.at[r]], add=True)
        return c
    lax.fori_loop(0, _ROWS, _msg_scat, 0)
    plsc.subcore_barrier()

    pltpu.sync_copy(s_s.at[pl.ds(base, _PT)], sowno)

    def _score(i, c):
        sl = pl.ds(i * 16, 16)
        sc = diso[sl] * sowno[sl] + selfo[sl]
        gid = base + i * 16 + lax.iota(jnp.int32, 16)
        sco[sl] = jnp.where(gid < _N, sc, -1e30)
        return c
    lax.fori_loop(0, _PT // 16, _score, 0)
    pltpu.sync_copy(sco, scores.at[pl.ds(base, _PT)])


_sc_call = functools.partial(
    pl.kernel,
    out_type=jax.ShapeDtypeStruct((_NPN,), jnp.float32),
    mesh=plsc.VectorSubcoreMesh(core_axis_name="c", subcore_axis_name="s",
                                num_cores=1),
    compiler_params=pltpu.CompilerParams(needs_layout_passes=False),
    scratch_types=[
        pltpu.VMEM((_ROWS, 128), jnp.int32),
        pltpu.VMEM((_EPAD,), jnp.int32),
        pltpu.VMEM((_EPAD,), jnp.float32),
        pltpu.VMEM((_NPN,), jnp.float32),
        pltpu.VMEM((_PT,), jnp.float32),
        pltpu.VMEM((_PT,), jnp.float32),
        pltpu.VMEM((_PT,), jnp.float32),
        pltpu.VMEM((_PT,), jnp.float32),
        pltpu.VMEM((_PT,), jnp.float32),
        pltpu.VMEM((_PT,), jnp.float32),
        pltpu.VMEM((_PT,), jnp.float32),
        pltpu.VMEM((_PT,), jnp.float32),
        pltpu.VMEM_SHARED((_NPN,), jnp.float32),
        pltpu.VMEM_SHARED((_NPN,), jnp.float32),
        pltpu.VMEM_SHARED((_NPN,), jnp.float32),
    ],
)(_sc_body)


def _pad_edges(idx, flat):
    per = _E // _NT
    npad = _EPAD - per
    pad = 10000 + (jnp.arange(_NT * npad, dtype=jnp.int32) % 240)
    pad = pad.reshape(_NT, npad)
    out = jnp.concatenate([idx.reshape(_NT, per), pad], axis=1)
    return out.reshape(_NT * _EPAD) if flat else out.reshape(_NT, _ROWS, 128)


def kernel(x, edge_index, batch, W, b):
    h = _tc_matvec(x, W).reshape(_NPN)
    srcp = _pad_edges(edge_index[0], flat=True)
    dstp = _pad_edges(edge_index[1], flat=False)
    sc_raw = _sc_call(h, srcp, dstp)
    batchp = jnp.concatenate(
        [batch, jnp.full((_NPN - _N,), -1, jnp.int32)]).reshape(_NPN, 1)
    scores = _tc_softmax(sc_raw.reshape(_NPN, 1), batchp)
    scores = scores[:_N].reshape(_N, 1)
    perm = jnp.arange(_N, dtype=jnp.int32)
    return (x, edge_index, batch, perm, scores)


# speedup vs baseline: 80.8065x; 1.3313x over previous
import functools

import jax
import jax.numpy as jnp
from jax import lax
from jax.experimental import pallas as pl
from jax.experimental.pallas import tpu as pltpu
from jax.experimental.pallas import tpu_sc as plsc

_N = 10000
_E = 320000
_D = 128
_G = 64

_NT = 16
_PT = 640
_NPN = _NT * _PT
_EPT = _E // _NT


def _matvec_body(x_ref, w_ref, o_ref):
    i = pl.program_id(0)
    h = jnp.dot(x_ref[...], w_ref[...], preferred_element_type=jnp.float32)
    gid = i * _PT + lax.broadcasted_iota(jnp.int32, (_PT, 1), 0)
    o_ref[...] = jnp.where(gid < _N, h, 0.0)


def _tc_matvec(x, W):
    return pl.pallas_call(
        _matvec_body,
        grid=(_NPN // _PT,),
        in_specs=[
            pl.BlockSpec((_PT, _D), lambda i: (i, 0)),
            pl.BlockSpec((_D, 1), lambda i: (0, 0)),
        ],
        out_specs=pl.BlockSpec((_PT, 1), lambda i: (i, 0)),
        out_shape=jax.ShapeDtypeStruct((_NPN, 1), jnp.float32),
    )(x, W)


def _softmax_body(s_ref, b_ref, o_ref):
    s = s_ref[...]
    b = b_ref[...]
    gmax = jnp.max(s)
    gids = lax.broadcasted_iota(jnp.int32, (_N, _G), 1)
    oh = (b == gids).astype(jnp.float32)
    ex = jnp.exp(s - gmax)
    ssum = lax.dot_general(oh, ex, (((0,), (0,)), ((), ())),
                           preferred_element_type=jnp.float32)
    ssum_i = jnp.dot(oh, ssum, preferred_element_type=jnp.float32)
    o_ref[...] = ex / (ssum_i + 1e-16)


def _tc_softmax(sc_raw, batch2):
    return pl.pallas_call(
        _softmax_body,
        out_shape=jax.ShapeDtypeStruct((_N, 1), jnp.float32),
    )(sc_raw, batch2)


def _rsqrt16(d):
    i = plsc.bitcast(d, jnp.int32)
    i = jnp.int32(0x5F3759DF) - lax.shift_right_logical(i, 1)
    y = plsc.bitcast(i, jnp.float32)
    for _ in range(3):
        y = y * (1.5 - 0.5 * d * y * y)
    return y


def _sc_body(hp, srcf, dstf, scores,
             dstv, srcv, valv, hown, dego, diso, selfo, gowno, sowno, sco,
             zer, deg_s, g_s, s_s):
    s = lax.axis_index("s")
    base = s * _PT
    ebase = s * _EPT

    def _zero(i, c):
        zer[pl.ds(i * 16, 16)] = jnp.zeros((16,), jnp.float32)
        return c
    lax.fori_loop(0, _PT // 16, _zero, 0)

    pltpu.sync_copy(zer, deg_s.at[pl.ds(base, _PT)])
    pltpu.sync_copy(zer, s_s.at[pl.ds(base, _PT)])
    pltpu.sync_copy(dstf.at[pl.ds(ebase, _EPT)], dstv)
    pltpu.sync_copy(srcf.at[pl.ds(ebase, _EPT)], srcv)
    pltpu.sync_copy(hp.at[pl.ds(base, _PT)], hown)

    ones16 = jnp.full((16,), 1.0, jnp.float32)

    def _fill_ones(r, c):
        for k in range(8):
            valv[pl.ds(r * 128 + k * 16, 16)] = ones16
        return c
    lax.fori_loop(0, _EPT // 128, _fill_ones, 0)
    for k in range(2):
        valv[pl.ds((_EPT // 128) * 128 + k * 16, 16)] = ones16

    plsc.subcore_barrier()

    pltpu.sync_copy(valv, deg_s.at[dstv], add=True)
    plsc.subcore_barrier()

    pltpu.sync_copy(deg_s.at[pl.ds(base, _PT)], dego)

    def _norm(i, c):
        sl = pl.ds(i * 16, 16)
        d = dego[sl] + 1.0
        y = _rsqrt16(d)
        diso[sl] = y
        selfo[sl] = hown[sl] / d
        gowno[sl] = y * hown[sl]
        return c
    lax.fori_loop(0, _PT // 16, _norm, 0)

    pltpu.sync_copy(gowno, g_s.at[pl.ds(base, _PT)])
    plsc.subcore_barrier()

    pltpu.sync_copy(g_s.at[srcv], valv)
    pltpu.sync_copy(valv, s_s.at[dstv], add=True)
    plsc.subcore_barrier()

    pltpu.sync_copy(s_s.at[pl.ds(base, _PT)], sowno)

    def _score(i, c):
        sl = pl.ds(i * 16, 16)
        sco[sl] = diso[sl] * sowno[sl] + selfo[sl]
        return c
    lax.fori_loop(0, _PT // 16, _score, 0)
    pltpu.sync_copy(sco, scores.at[pl.ds(base, _PT)])


_sc_call = functools.partial(
    pl.kernel,
    out_type=jax.ShapeDtypeStruct((_NPN,), jnp.float32),
    mesh=plsc.VectorSubcoreMesh(core_axis_name="c", subcore_axis_name="s",
                                num_cores=1),
    compiler_params=pltpu.CompilerParams(needs_layout_passes=False),
    scratch_types=[
        pltpu.VMEM((_EPT,), jnp.int32),
        pltpu.VMEM((_EPT,), jnp.int32),
        pltpu.VMEM((_EPT,), jnp.float32),
        pltpu.VMEM((_PT,), jnp.float32),
        pltpu.VMEM((_PT,), jnp.float32),
        pltpu.VMEM((_PT,), jnp.float32),
        pltpu.VMEM((_PT,), jnp.float32),
        pltpu.VMEM((_PT,), jnp.float32),
        pltpu.VMEM((_PT,), jnp.float32),
        pltpu.VMEM((_PT,), jnp.float32),
        pltpu.VMEM((_PT,), jnp.float32),
        pltpu.VMEM_SHARED((_NPN,), jnp.float32),
        pltpu.VMEM_SHARED((_NPN,), jnp.float32),
        pltpu.VMEM_SHARED((_NPN,), jnp.float32),
    ],
)(_sc_body)


def kernel(x, edge_index, batch, W, b):
    h = _tc_matvec(x, W).reshape(_NPN)
    sc_raw = _sc_call(h, edge_index[0], edge_index[1])
    scores = _tc_softmax(sc_raw[:_N].reshape(_N, 1), batch.reshape(_N, 1))
    perm = jnp.arange(_N, dtype=jnp.int32)
    return (x, edge_index, batch, perm, scores)


# speedup vs baseline: 94.5554x; 1.1701x over previous
import functools

import jax
import jax.numpy as jnp
from jax import lax
from jax.experimental import pallas as pl
from jax.experimental.pallas import tpu as pltpu
from jax.experimental.pallas import tpu_sc as plsc

_N = 10000
_E = 320000
_D = 128
_G = 64

_NT = 16
_PT = 640
_NPN = _NT * _PT
_EPT = _E // _NT
_TBL = 128 * _G


def _matvec_body(x_ref, w_ref, o_ref):
    i = pl.program_id(0)
    h = jnp.dot(x_ref[...], w_ref[...], preferred_element_type=jnp.float32)
    gid = i * _PT + lax.broadcasted_iota(jnp.int32, (_PT, 1), 0)
    o_ref[...] = jnp.where(gid < _N, h, 0.0)


def _tc_matvec(x, W):
    return pl.pallas_call(
        _matvec_body,
        grid=(_NPN // _PT,),
        in_specs=[
            pl.BlockSpec((_PT, _D), lambda i: (i, 0)),
            pl.BlockSpec((_D, 1), lambda i: (0, 0)),
        ],
        out_specs=pl.BlockSpec((_PT, 1), lambda i: (i, 0)),
        out_shape=jax.ShapeDtypeStruct((_NPN, 1), jnp.float32),
    )(x, W)


def _rsqrt16(d):
    i = plsc.bitcast(d, jnp.int32)
    i = jnp.int32(0x5F3759DF) - lax.shift_right_logical(i, 1)
    y = plsc.bitcast(i, jnp.float32)
    for _ in range(3):
        y = y * (1.5 - 0.5 * d * y * y)
    return y


def _sc_body(hp, srcf, dstf, batchp, scores,
             dstv, srcv, valv, hown, dego, diso, selfo, gowno, sowno, sco,
             batchof, sltv, exo, outo, zer, mx, redv, tblv, ssumv,
             deg_s, g_s, s_s, red_s, tbl_s, ssum_s):
    s = lax.axis_index("s")
    base = s * _PT
    ebase = s * _EPT

    def _zero(i, c):
        zer[pl.ds(i * 16, 16)] = jnp.zeros((16,), jnp.float32)
        return c
    lax.fori_loop(0, _PT // 16, _zero, 0)

    pltpu.sync_copy(zer, deg_s.at[pl.ds(base, _PT)])
    pltpu.sync_copy(zer, s_s.at[pl.ds(base, _PT)])
    pltpu.sync_copy(zer.at[pl.ds(0, _TBL // _NT)],
                    tbl_s.at[pl.ds(s * (_TBL // _NT), _TBL // _NT)])
    pltpu.sync_copy(dstf.at[pl.ds(ebase, _EPT)], dstv)
    pltpu.sync_copy(srcf.at[pl.ds(ebase, _EPT)], srcv)
    pltpu.sync_copy(hp.at[pl.ds(base, _PT)], hown)
    pltpu.sync_copy(batchp.at[pl.ds(base, _PT)], batchof)

    ones16 = jnp.full((16,), 1.0, jnp.float32)

    def _fill_ones(r, c):
        for k in range(8):
            valv[pl.ds(r * 128 + k * 16, 16)] = ones16
        return c
    lax.fori_loop(0, _EPT // 128, _fill_ones, 0)
    for k in range(2):
        valv[pl.ds((_EPT // 128) * 128 + k * 16, 16)] = ones16

    plsc.subcore_barrier()

    pltpu.sync_copy(valv, deg_s.at[dstv], add=True)
    plsc.subcore_barrier()

    pltpu.sync_copy(deg_s.at[pl.ds(base, _PT)], dego)

    def _norm(i, c):
        sl = pl.ds(i * 16, 16)
        d = dego[sl] + 1.0
        y = _rsqrt16(d)
        diso[sl] = y
        selfo[sl] = hown[sl] / d
        gowno[sl] = y * hown[sl]
        return c
    lax.fori_loop(0, _PT // 16, _norm, 0)

    pltpu.sync_copy(gowno, g_s.at[pl.ds(base, _PT)])
    plsc.subcore_barrier()

    pltpu.sync_copy(g_s.at[srcv], valv)
    pltpu.sync_copy(valv, s_s.at[dstv], add=True)
    plsc.subcore_barrier()

    pltpu.sync_copy(s_s.at[pl.ds(base, _PT)], sowno)

    def _score(i, m):
        sl = pl.ds(i * 16, 16)
        sc = diso[sl] * sowno[sl] + selfo[sl]
        sco[sl] = sc
        gid = base + i * 16 + lax.iota(jnp.int32, 16)
        return jnp.maximum(m, jnp.where(gid < _N, sc, -1e30))
    m = lax.fori_loop(0, _PT // 16, _score, jnp.full((16,), -1e30, jnp.float32))
    mx[...] = m
    pltpu.sync_copy(mx, red_s.at[pl.ds(s * 16, 16)])
    plsc.subcore_barrier()
    pltpu.sync_copy(red_s, redv)
    m2 = jnp.full((16,), -1e30, jnp.float32)
    for k in range(_NT):
        m2 = jnp.maximum(m2, redv[pl.ds(k * 16, 16)])
    gmax = jnp.max(m2)

    def _exp(i, c):
        sl = pl.ds(i * 16, 16)
        gid = base + i * 16 + lax.iota(jnp.int32, 16)
        ex = jnp.exp(sco[sl] - gmax)
        exo[sl] = jnp.where(gid < _N, ex, 0.0)
        lane = (i % 8) * 16 + lax.iota(jnp.int32, 16)
        sltv[sl] = lane * _G + batchof[sl]
        return c
    lax.fori_loop(0, _PT // 16, _exp, 0)

    pltpu.sync_copy(exo, tbl_s.at[sltv], add=True)
    plsc.subcore_barrier()

    @pl.when(s == 0)
    def _():
        pltpu.sync_copy(tbl_s, tblv)

        def _red(k, acc):
            return tuple(
                acc[j] + tblv[pl.ds(k * _G + j * 16, 16)] for j in range(4))
        acc = lax.fori_loop(
            0, 128, _red, tuple(jnp.zeros((16,), jnp.float32) for _ in range(4)))
        for j in range(4):
            ssumv[pl.ds(j * 16, 16)] = acc[j]
        pltpu.sync_copy(ssumv, ssum_s)
    plsc.subcore_barrier()
    pltpu.sync_copy(ssum_s, ssumv)

    def _norm_out(i, c):
        sl = pl.ds(i * 16, 16)
        ss = plsc.load_gather(ssumv, [batchof[sl]])
        outo[sl] = exo[sl] / (ss + 1e-16)
        return c
    lax.fori_loop(0, _PT // 16, _norm_out, 0)
    pltpu.sync_copy(outo, scores.at[pl.ds(base, _PT)])


_sc_call = functools.partial(
    pl.kernel,
    out_type=jax.ShapeDtypeStruct((_NPN,), jnp.float32),
    mesh=plsc.VectorSubcoreMesh(core_axis_name="c", subcore_axis_name="s",
                                num_cores=1),
    compiler_params=pltpu.CompilerParams(needs_layout_passes=False),
    scratch_types=[
        pltpu.VMEM((_EPT,), jnp.int32),
        pltpu.VMEM((_EPT,), jnp.int32),
        pltpu.VMEM((_EPT,), jnp.float32),
        pltpu.VMEM((_PT,), jnp.float32),
        pltpu.VMEM((_PT,), jnp.float32),
        pltpu.VMEM((_PT,), jnp.float32),
        pltpu.VMEM((_PT,), jnp.float32),
        pltpu.VMEM((_PT,), jnp.float32),
        pltpu.VMEM((_PT,), jnp.float32),
        pltpu.VMEM((_PT,), jnp.float32),
        pltpu.VMEM((_PT,), jnp.int32),
        pltpu.VMEM((_PT,), jnp.int32),
        pltpu.VMEM((_PT,), jnp.float32),
        pltpu.VMEM((_PT,), jnp.float32),
        pltpu.VMEM((_PT,), jnp.float32),
        pltpu.VMEM((16,), jnp.float32),
        pltpu.VMEM((_NT * 16,), jnp.float32),
        pltpu.VMEM((_TBL,), jnp.float32),
        pltpu.VMEM((_G,), jnp.float32),
        pltpu.VMEM_SHARED((_NPN,), jnp.float32),
        pltpu.VMEM_SHARED((_NPN,), jnp.float32),
        pltpu.VMEM_SHARED((_NPN,), jnp.float32),
        pltpu.VMEM_SHARED((_NT * 16,), jnp.float32),
        pltpu.VMEM_SHARED((_TBL,), jnp.float32),
        pltpu.VMEM_SHARED((_G,), jnp.float32),
    ],
)(_sc_body)


def kernel(x, edge_index, batch, W, b):
    h = _tc_matvec(x, W).reshape(_NPN)
    batchp = jnp.concatenate([batch, jnp.full((_NPN - _N,), _G - 1, jnp.int32)])
    scores = _sc_call(h, edge_index[0], edge_index[1], batchp)
    scores = scores[:_N].reshape(_N, 1)
    perm = jnp.arange(_N, dtype=jnp.int32)
    return (x, edge_index, batch, perm, scores)


# speedup vs baseline: 98.3746x; 1.0404x over previous
import functools

import jax
import jax.numpy as jnp
from jax import lax
from jax.experimental import pallas as pl
from jax.experimental.pallas import tpu as pltpu
from jax.experimental.pallas import tpu_sc as plsc

_N = 10000
_E = 320000
_D = 128
_G = 64

_NT = 16
_PT = 640
_NPN = _NT * _PT
_EPT = _E // _NT
_TBL = 128 * _G


def _matvec_body(x_ref, w_ref, o_ref):
    i = pl.program_id(0)
    h = jnp.dot(x_ref[...], w_ref[...], preferred_element_type=jnp.float32)
    gid = i * _PT + lax.broadcasted_iota(jnp.int32, (_PT, 1), 0)
    o_ref[...] = jnp.where(gid < _N, h, 0.0)


def _tc_matvec(x, W):
    return pl.pallas_call(
        _matvec_body,
        grid=(_NPN // _PT,),
        in_specs=[
            pl.BlockSpec((_PT, _D), lambda i: (i, 0)),
            pl.BlockSpec((_D, 1), lambda i: (0, 0)),
        ],
        out_specs=pl.BlockSpec((_PT, 1), lambda i: (i, 0)),
        out_shape=jax.ShapeDtypeStruct((_NPN, 1), jnp.float32),
    )(x, W)


def _rsqrt16(d):
    i = plsc.bitcast(d, jnp.int32)
    i = jnp.int32(0x5F3759DF) - lax.shift_right_logical(i, 1)
    y = plsc.bitcast(i, jnp.float32)
    for _ in range(3):
        y = y * (1.5 - 0.5 * d * y * y)
    return y


def _sc_body(hp, srcf, dstf, batchp, scores,
             dstv, srcv, valv, hown, dego, diso, selfo, gowno, sowno, sco,
             batchof, sltv, exo, outo, zer, mx, redv, tblv, ssumv,
             sem1, sem2, sem3,
             deg_s, g_s, s_s, red_s, tbl_s, ssum_s):
    s = lax.axis_index("s")
    base = s * _PT
    ebase = s * _EPT

    cp1 = pltpu.async_copy(dstf.at[pl.ds(ebase, _EPT)], dstv, sem1)
    cp2 = pltpu.async_copy(srcf.at[pl.ds(ebase, _EPT)], srcv, sem2)
    cp3 = pltpu.async_copy(hp.at[pl.ds(base, _PT)], hown, sem3)

    def _zero(i, c):
        zer[pl.ds(i * 16, 16)] = jnp.zeros((16,), jnp.float32)
        return c
    lax.fori_loop(0, _PT // 16, _zero, 0)

    pltpu.sync_copy(zer, deg_s.at[pl.ds(base, _PT)])
    pltpu.sync_copy(zer, s_s.at[pl.ds(base, _PT)])
    pltpu.sync_copy(zer.at[pl.ds(0, _TBL // _NT)],
                    tbl_s.at[pl.ds(s * (_TBL // _NT), _TBL // _NT)])

    sixty3 = jnp.full((16,), _G - 1, jnp.int32)

    @pl.when(s < _NT - 1)
    def _():
        pltpu.sync_copy(batchp.at[pl.ds(base, _PT)], batchof)

    @pl.when(s == _NT - 1)
    def _():
        for k in range(15):
            batchof[pl.ds(400 + k * 16, 16)] = sixty3
        pltpu.sync_copy(batchp.at[pl.ds(base, 400)],
                        batchof.at[pl.ds(0, 400)])

    ones16 = jnp.full((16,), 1.0, jnp.float32)

    def _fill_ones(r, c):
        for k in range(8):
            valv[pl.ds(r * 128 + k * 16, 16)] = ones16
        return c
    lax.fori_loop(0, _EPT // 128, _fill_ones, 0)
    for k in range(2):
        valv[pl.ds((_EPT // 128) * 128 + k * 16, 16)] = ones16

    cp1.wait()
    plsc.subcore_barrier()

    pltpu.sync_copy(valv, deg_s.at[dstv], add=True)
    plsc.subcore_barrier()

    pltpu.sync_copy(deg_s.at[pl.ds(base, _PT)], dego)
    cp3.wait()

    def _norm(i, c):
        sl = pl.ds(i * 16, 16)
        d = dego[sl] + 1.0
        y = _rsqrt16(d)
        diso[sl] = y
        selfo[sl] = hown[sl] / d
        gowno[sl] = y * hown[sl]
        return c
    lax.fori_loop(0, _PT // 16, _norm, 0)

    pltpu.sync_copy(gowno, g_s.at[pl.ds(base, _PT)])
    plsc.subcore_barrier()

    cp2.wait()
    pltpu.sync_copy(g_s.at[srcv], valv)
    pltpu.sync_copy(valv, s_s.at[dstv], add=True)
    plsc.subcore_barrier()

    pltpu.sync_copy(s_s.at[pl.ds(base, _PT)], sowno)

    def _score(i, m):
        sl = pl.ds(i * 16, 16)
        sc = diso[sl] * sowno[sl] + selfo[sl]
        sco[sl] = sc
        gid = base + i * 16 + lax.iota(jnp.int32, 16)
        return jnp.maximum(m, jnp.where(gid < _N, sc, -1e30))
    m = lax.fori_loop(0, _PT // 16, _score, jnp.full((16,), -1e30, jnp.float32))
    mx[...] = m
    pltpu.sync_copy(mx, red_s.at[pl.ds(s * 16, 16)])
    plsc.subcore_barrier()
    pltpu.sync_copy(red_s, redv)
    m2 = jnp.full((16,), -1e30, jnp.float32)
    for k in range(_NT):
        m2 = jnp.maximum(m2, redv[pl.ds(k * 16, 16)])
    gmax = jnp.max(m2)

    def _exp(i, c):
        sl = pl.ds(i * 16, 16)
        gid = base + i * 16 + lax.iota(jnp.int32, 16)
        ex = jnp.exp(sco[sl] - gmax)
        exo[sl] = jnp.where(gid < _N, ex, 0.0)
        lane = (i % 8) * 16 + lax.iota(jnp.int32, 16)
        sltv[sl] = lane * _G + batchof[sl]
        return c
    lax.fori_loop(0, _PT // 16, _exp, 0)

    pltpu.sync_copy(exo, tbl_s.at[sltv], add=True)
    plsc.subcore_barrier()

    @pl.when(s == 0)
    def _():
        pltpu.sync_copy(tbl_s, tblv)

        def _red(k, acc):
            return tuple(
                acc[j] + tblv[pl.ds(k * _G + j * 16, 16)] for j in range(4))
        acc = lax.fori_loop(
            0, 128, _red, tuple(jnp.zeros((16,), jnp.float32) for _ in range(4)))
        for j in range(4):
            ssumv[pl.ds(j * 16, 16)] = acc[j]
        pltpu.sync_copy(ssumv, ssum_s)
    plsc.subcore_barrier()
    pltpu.sync_copy(ssum_s, ssumv)

    def _norm_out(i, c):
        sl = pl.ds(i * 16, 16)
        ss = plsc.load_gather(ssumv, [batchof[sl]])
        outo[sl] = exo[sl] / (ss + 1e-16)
        return c
    lax.fori_loop(0, _PT // 16, _norm_out, 0)
    pltpu.sync_copy(outo, scores.at[pl.ds(base, _PT)])


_sc_call = functools.partial(
    pl.kernel,
    out_type=jax.ShapeDtypeStruct((_NPN,), jnp.float32),
    mesh=plsc.VectorSubcoreMesh(core_axis_name="c", subcore_axis_name="s",
                                num_cores=1),
    compiler_params=pltpu.CompilerParams(needs_layout_passes=False),
    scratch_types=[
        pltpu.VMEM((_EPT,), jnp.int32),
        pltpu.VMEM((_EPT,), jnp.int32),
        pltpu.VMEM((_EPT,), jnp.float32),
        pltpu.VMEM((_PT,), jnp.float32),
        pltpu.VMEM((_PT,), jnp.float32),
        pltpu.VMEM((_PT,), jnp.float32),
        pltpu.VMEM((_PT,), jnp.float32),
        pltpu.VMEM((_PT,), jnp.float32),
        pltpu.VMEM((_PT,), jnp.float32),
        pltpu.VMEM((_PT,), jnp.float32),
        pltpu.VMEM((_PT,), jnp.int32),
        pltpu.VMEM((_PT,), jnp.int32),
        pltpu.VMEM((_PT,), jnp.float32),
        pltpu.VMEM((_PT,), jnp.float32),
        pltpu.VMEM((_PT,), jnp.float32),
        pltpu.VMEM((16,), jnp.float32),
        pltpu.VMEM((_NT * 16,), jnp.float32),
        pltpu.VMEM((_TBL,), jnp.float32),
        pltpu.VMEM((_G,), jnp.float32),
        pltpu.SemaphoreType.DMA,
        pltpu.SemaphoreType.DMA,
        pltpu.SemaphoreType.DMA,
        pltpu.VMEM_SHARED((_NPN,), jnp.float32),
        pltpu.VMEM_SHARED((_NPN,), jnp.float32),
        pltpu.VMEM_SHARED((_NPN,), jnp.float32),
        pltpu.VMEM_SHARED((_NT * 16,), jnp.float32),
        pltpu.VMEM_SHARED((_TBL,), jnp.float32),
        pltpu.VMEM_SHARED((_G,), jnp.float32),
    ],
)(_sc_body)


def kernel(x, edge_index, batch, W, b):
    h = _tc_matvec(x, W).reshape(_NPN)
    scores = _sc_call(h, edge_index[0], edge_index[1], batch)
    scores = scores[:_N].reshape(_N, 1)
    perm = jnp.arange(_N, dtype=jnp.int32)
    return (x, edge_index, batch, perm, scores)


# speedup vs baseline: 98.4052x; 1.0003x over previous
import functools

import jax
import jax.numpy as jnp
from jax import lax
from jax.experimental import pallas as pl
from jax.experimental.pallas import tpu as pltpu
from jax.experimental.pallas import tpu_sc as plsc

_N = 10000
_E = 320000
_D = 128
_G = 64

_NT = 16
_PT = 640
_NPN = _NT * _PT
_EPT = _E // _NT
_TBL = 128 * _G


def _matvec_body(x_ref, w_ref, o_ref):
    i = pl.program_id(0)
    h = jnp.dot(x_ref[...], w_ref[...], preferred_element_type=jnp.float32)
    gid = i * _PT + lax.broadcasted_iota(jnp.int32, (_PT, 1), 0)
    o_ref[...] = jnp.where(gid < _N, h, 0.0)


def _tc_matvec(x, W):
    return pl.pallas_call(
        _matvec_body,
        grid=(_NPN // _PT,),
        in_specs=[
            pl.BlockSpec((_PT, _D), lambda i: (i, 0)),
            pl.BlockSpec((_D, 1), lambda i: (0, 0)),
        ],
        out_specs=pl.BlockSpec((_PT, 1), lambda i: (i, 0)),
        out_shape=jax.ShapeDtypeStruct((_NPN, 1), jnp.float32),
    )(x, W)


def _rsqrt16(d):
    i = plsc.bitcast(d, jnp.int32)
    i = jnp.int32(0x5F3759DF) - lax.shift_right_logical(i, 1)
    y = plsc.bitcast(i, jnp.float32)
    for _ in range(3):
        y = y * (1.5 - 0.5 * d * y * y)
    return y


def _sc_body(hp, srcf, dstf, batchp, scores,
             dstv, srcv, valv, hown, dego, diso, selfo, gowno, sowno, sco,
             batchof, sltv, exo, outo, zer, mx, redv, tblv, ssumv,
             sem1, sem2, sem3,
             deg_s, g_s, s_s, red_s, tbl_s, ssum_s):
    s = lax.axis_index("s")
    base = s * _PT
    ebase = s * _EPT

    cp1 = pltpu.async_copy(dstf.at[pl.ds(ebase, _EPT)], dstv, sem1)
    cp2 = pltpu.async_copy(srcf.at[pl.ds(ebase, _EPT)], srcv, sem2)
    cp3 = pltpu.async_copy(hp.at[pl.ds(base, _PT)], hown, sem3)

    def _zero(i, c):
        zer[pl.ds(i * 16, 16)] = jnp.zeros((16,), jnp.float32)
        return c
    lax.fori_loop(0, _PT // 16, _zero, 0)

    pltpu.sync_copy(zer, deg_s.at[pl.ds(base, _PT)])
    pltpu.sync_copy(zer, s_s.at[pl.ds(base, _PT)])
    pltpu.sync_copy(zer.at[pl.ds(0, _TBL // _NT)],
                    tbl_s.at[pl.ds(s * (_TBL // _NT), _TBL // _NT)])

    sixty3 = jnp.full((16,), _G - 1, jnp.int32)

    @pl.when(s < _NT - 1)
    def _():
        pltpu.sync_copy(batchp.at[pl.ds(base, _PT)], batchof)

    @pl.when(s == _NT - 1)
    def _():
        for k in range(15):
            batchof[pl.ds(400 + k * 16, 16)] = sixty3
        pltpu.sync_copy(batchp.at[pl.ds(base, 400)],
                        batchof.at[pl.ds(0, 400)])

    ones16 = jnp.full((16,), 1.0, jnp.float32)

    def _fill_ones(r, c):
        for k in range(8):
            valv[pl.ds(r * 128 + k * 16, 16)] = ones16
        return c
    lax.fori_loop(0, _EPT // 128, _fill_ones, 0)
    for k in range(2):
        valv[pl.ds((_EPT // 128) * 128 + k * 16, 16)] = ones16

    cp1.wait()
    plsc.subcore_barrier()

    pltpu.sync_copy(valv, deg_s.at[dstv], add=True)
    plsc.subcore_barrier()

    pltpu.sync_copy(deg_s.at[pl.ds(base, _PT)], dego)
    cp3.wait()

    def _norm(i, c):
        sl = pl.ds(i * 16, 16)
        d = dego[sl] + 1.0
        y = _rsqrt16(d)
        diso[sl] = y
        selfo[sl] = hown[sl] / d
        gowno[sl] = y * hown[sl]
        return c
    lax.fori_loop(0, _PT // 16, _norm, 0)

    pltpu.sync_copy(gowno, g_s.at[pl.ds(base, _PT)])
    plsc.subcore_barrier()

    cp2.wait()
    eh = _EPT // 2
    ga = pltpu.async_copy(g_s.at[srcv.at[pl.ds(0, eh)]],
                          valv.at[pl.ds(0, eh)], sem1)
    gb = pltpu.async_copy(g_s.at[srcv.at[pl.ds(eh, eh)]],
                          valv.at[pl.ds(eh, eh)], sem2)
    ga.wait()
    sa = pltpu.async_copy(valv.at[pl.ds(0, eh)],
                          s_s.at[dstv.at[pl.ds(0, eh)]], sem3, add=True)
    gb.wait()
    sb = pltpu.async_copy(valv.at[pl.ds(eh, eh)],
                          s_s.at[dstv.at[pl.ds(eh, eh)]], sem1, add=True)
    sa.wait()
    sb.wait()
    plsc.subcore_barrier()

    pltpu.sync_copy(s_s.at[pl.ds(base, _PT)], sowno)

    def _score(i, m):
        sl = pl.ds(i * 16, 16)
        sc = diso[sl] * sowno[sl] + selfo[sl]
        sco[sl] = sc
        gid = base + i * 16 + lax.iota(jnp.int32, 16)
        return jnp.maximum(m, jnp.where(gid < _N, sc, -1e30))
    m = lax.fori_loop(0, _PT // 16, _score, jnp.full((16,), -1e30, jnp.float32))
    mx[...] = m
    pltpu.sync_copy(mx, red_s.at[pl.ds(s * 16, 16)])
    plsc.subcore_barrier()
    pltpu.sync_copy(red_s, redv)
    m2 = jnp.full((16,), -1e30, jnp.float32)
    for k in range(_NT):
        m2 = jnp.maximum(m2, redv[pl.ds(k * 16, 16)])
    gmax = jnp.max(m2)

    def _exp(i, c):
        sl = pl.ds(i * 16, 16)
        gid = base + i * 16 + lax.iota(jnp.int32, 16)
        ex = jnp.exp(sco[sl] - gmax)
        exo[sl] = jnp.where(gid < _N, ex, 0.0)
        lane = (i % 8) * 16 + lax.iota(jnp.int32, 16)
        sltv[sl] = lane * _G + batchof[sl]
        return c
    lax.fori_loop(0, _PT // 16, _exp, 0)

    pltpu.sync_copy(exo, tbl_s.at[sltv], add=True)
    plsc.subcore_barrier()

    @pl.when(s == 0)
    def _():
        pltpu.sync_copy(tbl_s, tblv)

        def _red(k, acc):
            return tuple(
                acc[j] + tblv[pl.ds(k * _G + j * 16, 16)] for j in range(4))
        acc = lax.fori_loop(
            0, 128, _red, tuple(jnp.zeros((16,), jnp.float32) for _ in range(4)))
        for j in range(4):
            ssumv[pl.ds(j * 16, 16)] = acc[j]
        pltpu.sync_copy(ssumv, ssum_s)
    plsc.subcore_barrier()
    pltpu.sync_copy(ssum_s, ssumv)

    def _norm_out(i, c):
        sl = pl.ds(i * 16, 16)
        ss = plsc.load_gather(ssumv, [batchof[sl]])
        outo[sl] = exo[sl] / (ss + 1e-16)
        return c
    lax.fori_loop(0, _PT // 16, _norm_out, 0)
    pltpu.sync_copy(outo, scores.at[pl.ds(base, _PT)])


_sc_call = functools.partial(
    pl.kernel,
    out_type=jax.ShapeDtypeStruct((_NPN,), jnp.float32),
    mesh=plsc.VectorSubcoreMesh(core_axis_name="c", subcore_axis_name="s",
                                num_cores=1),
    compiler_params=pltpu.CompilerParams(needs_layout_passes=False),
    scratch_types=[
        pltpu.VMEM((_EPT,), jnp.int32),
        pltpu.VMEM((_EPT,), jnp.int32),
        pltpu.VMEM((_EPT,), jnp.float32),
        pltpu.VMEM((_PT,), jnp.float32),
        pltpu.VMEM((_PT,), jnp.float32),
        pltpu.VMEM((_PT,), jnp.float32),
        pltpu.VMEM((_PT,), jnp.float32),
        pltpu.VMEM((_PT,), jnp.float32),
        pltpu.VMEM((_PT,), jnp.float32),
        pltpu.VMEM((_PT,), jnp.float32),
        pltpu.VMEM((_PT,), jnp.int32),
        pltpu.VMEM((_PT,), jnp.int32),
        pltpu.VMEM((_PT,), jnp.float32),
        pltpu.VMEM((_PT,), jnp.float32),
        pltpu.VMEM((_PT,), jnp.float32),
        pltpu.VMEM((16,), jnp.float32),
        pltpu.VMEM((_NT * 16,), jnp.float32),
        pltpu.VMEM((_TBL,), jnp.float32),
        pltpu.VMEM((_G,), jnp.float32),
        pltpu.SemaphoreType.DMA,
        pltpu.SemaphoreType.DMA,
        pltpu.SemaphoreType.DMA,
        pltpu.VMEM_SHARED((_NPN,), jnp.float32),
        pltpu.VMEM_SHARED((_NPN,), jnp.float32),
        pltpu.VMEM_SHARED((_NPN,), jnp.float32),
        pltpu.VMEM_SHARED((_NT * 16,), jnp.float32),
        pltpu.VMEM_SHARED((_TBL,), jnp.float32),
        pltpu.VMEM_SHARED((_G,), jnp.float32),
    ],
)(_sc_body)


def kernel(x, edge_index, batch, W, b):
    h = _tc_matvec(x, W).reshape(_NPN)
    scores = _sc_call(h, edge_index[0], edge_index[1], batch)
    scores = scores[:_N].reshape(_N, 1)
    perm = jnp.arange(_N, dtype=jnp.int32)
    return (x, edge_index, batch, perm, scores)
